# baseline (speedup 1.0000x reference)
import functools

import jax
import jax.numpy as jnp
from jax import lax
from jax.experimental import pallas as pl
from jax.experimental.pallas import tpu as pltpu
from jax.experimental.pallas import tpu_sc as plsc

NC = 2
NS = 16
L = 16
NW = NC * NS

B = 16384
D = 128
BPW = B // NW
CH = 128
NCHUNK = BPW // CH

N_NEWS_ROWS = 100000
NT_PW = N_NEWS_ROWS // NW
CCH = 125
NCC = NT_PW // CCH


def _sc_body(user_ref, news_ref, ut_ref, nt_ref, scores_out,
             uidx, nidx, ubuf0, ubuf1, nbuf0, nbuf1, scores, usem, nsem):
    ubufs = (ubuf0, ubuf1)
    nbufs = (nbuf0, nbuf1)
    wid = lax.axis_index("s") * NC + lax.axis_index("c")
    base = wid * BPW

    for c in range(NCHUNK):
        pltpu.sync_copy(user_ref.at[pl.ds(base + c * CH, CH)], uidx.at[c])
        pltpu.sync_copy(news_ref.at[pl.ds(base + c * CH, CH)], nidx.at[c])

    uh = [None] * NCHUNK
    nh = [None] * NCHUNK
    uh[0] = pltpu.async_copy(ut_ref.at[uidx.at[0]], ubufs[0], usem)
    nh[0] = pltpu.async_copy(nt_ref.at[nidx.at[0]], nbufs[0], nsem)

    lanes = lax.iota(jnp.int32, L)

    for c in range(NCHUNK):
        cur = c % 2
        uh[c].wait()
        nh[c].wait()
        if c + 1 < NCHUNK:
            nxt = (c + 1) % 2
            uh[c + 1] = pltpu.async_copy(ut_ref.at[uidx.at[c + 1]], ubufs[nxt], usem)
            nh[c + 1] = pltpu.async_copy(nt_ref.at[nidx.at[c + 1]], nbufs[nxt], nsem)

        def grp_body(g, _, cur=cur, c=c):
            vec = jnp.zeros((L,), jnp.float32)
            for r in range(L):
                i = g * L + r
                acc = ubufs[cur][i, pl.ds(0, L)] * nbufs[cur][i, pl.ds(0, L)]
                for j in range(1, D // L):
                    acc = acc + (ubufs[cur][i, pl.ds(j * L, L)]
                                 * nbufs[cur][i, pl.ds(j * L, L)])
                s = jnp.sum(acc)
                vec = jnp.where(lanes == r, s, vec)
            scores[pl.ds(c * CH + g * L, L)] = vec
            return 0

        lax.fori_loop(0, CH // L, grp_body, 0)

    pltpu.sync_copy(scores, scores_out.at[pl.ds(base, BPW)])


CPB = 4000


def _tc_copy_body(x_ref, o_ref):
    o_ref[...] = x_ref[...]


@jax.jit
def _tc_copy(x):
    return pl.pallas_call(
        _tc_copy_body,
        grid=(x.shape[0] // CPB,),
        in_specs=[pl.BlockSpec((CPB, D), lambda i: (i, 0))],
        out_specs=pl.BlockSpec((CPB, D), lambda i: (i, 0)),
        out_shape=jax.ShapeDtypeStruct(x.shape, x.dtype),
    )(x)


@jax.jit
def _scores_and_news(user, news, user_table, news_table):
    mesh = plsc.VectorSubcoreMesh(core_axis_name="c", subcore_axis_name="s",
                                  num_cores=NC, num_subcores=NS)
    call = functools.partial(
        pl.kernel,
        out_type=jax.ShapeDtypeStruct((B,), jnp.float32),
        mesh=mesh,
        compiler_params=pltpu.CompilerParams(needs_layout_passes=False,
                                             use_tc_tiling_on_sc=False),
        scratch_types=[
            pltpu.VMEM((NCHUNK, CH), jnp.int32),
            pltpu.VMEM((NCHUNK, CH), jnp.int32),
            pltpu.VMEM((CH, D), jnp.float32),
            pltpu.VMEM((CH, D), jnp.float32),
            pltpu.VMEM((CH, D), jnp.float32),
            pltpu.VMEM((CH, D), jnp.float32),
            pltpu.VMEM((BPW,), jnp.float32),
            pltpu.SemaphoreType.DMA,
            pltpu.SemaphoreType.DMA,
        ],
    )(_sc_body)
    return call(user.astype(jnp.int32), news.astype(jnp.int32),
                user_table, news_table)


def kernel(user, news, user_table, news_table):
    scores = _scores_and_news(user, news, user_table, news_table)
    one = lax.optimization_barrier(jnp.float32(1.0))
    ut = _tc_copy(user_table)
    nt = news_table * one
    return (ut, nt, scores)


# speedup vs baseline: 1.0656x; 1.0656x over previous
import functools

import jax
import jax.numpy as jnp
from jax import lax
from jax.experimental import pallas as pl
from jax.experimental.pallas import tpu as pltpu
from jax.experimental.pallas import tpu_sc as plsc

NC = 2
NS = 16
L = 16
NW = NC * NS

B = 16384
D = 128
BPW = B // NW
CH = 128
NCHUNK = BPW // CH

N_NEWS_ROWS = 100000
NT_PW = N_NEWS_ROWS // NW
CCH = 125
NCC = NT_PW // CCH


def _sc_body(user_ref, news_ref, ut_ref, nt_ref, scores_out,
             uidx, nidx, ubuf0, ubuf1, nbuf0, nbuf1, scores, usem, nsem):
    ubufs = (ubuf0, ubuf1)
    nbufs = (nbuf0, nbuf1)
    wid = lax.axis_index("s") * NC + lax.axis_index("c")
    base = wid * BPW

    ih = pltpu.async_copy(user_ref.at[pl.ds(base, BPW)], uidx, usem)
    jh = pltpu.async_copy(news_ref.at[pl.ds(base, BPW)], nidx, nsem)
    ih.wait()
    jh.wait()

    uh = [None] * NCHUNK
    nh = [None] * NCHUNK
    uh[0] = pltpu.async_copy(ut_ref.at[uidx.at[pl.ds(0, CH)]], ubufs[0], usem)
    nh[0] = pltpu.async_copy(nt_ref.at[nidx.at[pl.ds(0, CH)]], nbufs[0], nsem)

    lanes = lax.iota(jnp.int32, L)

    for c in range(NCHUNK):
        cur = c % 2
        uh[c].wait()
        nh[c].wait()
        if c + 1 < NCHUNK:
            nxt = (c + 1) % 2
            uh[c + 1] = pltpu.async_copy(
                ut_ref.at[uidx.at[pl.ds((c + 1) * CH, CH)]], ubufs[nxt], usem)
            nh[c + 1] = pltpu.async_copy(
                nt_ref.at[nidx.at[pl.ds((c + 1) * CH, CH)]], nbufs[nxt], nsem)

        def grp_body(g, _, cur=cur, c=c):
            vec = jnp.zeros((L,), jnp.float32)
            for r in range(L):
                i = g * L + r
                acc = ubufs[cur][i, pl.ds(0, L)] * nbufs[cur][i, pl.ds(0, L)]
                for j in range(1, D // L):
                    acc = acc + (ubufs[cur][i, pl.ds(j * L, L)]
                                 * nbufs[cur][i, pl.ds(j * L, L)])
                s = jnp.sum(acc)
                vec = jnp.where(lanes == r, s, vec)
            scores[pl.ds(c * CH + g * L, L)] = vec
            return 0

        lax.fori_loop(0, CH // L, grp_body, 0)

    pltpu.sync_copy(scores, scores_out.at[pl.ds(base, BPW)])


@jax.jit
def _scores_and_news(user, news, user_table, news_table):
    mesh = plsc.VectorSubcoreMesh(core_axis_name="c", subcore_axis_name="s",
                                  num_cores=NC, num_subcores=NS)
    call = functools.partial(
        pl.kernel,
        out_type=jax.ShapeDtypeStruct((B,), jnp.float32),
        mesh=mesh,
        compiler_params=pltpu.CompilerParams(needs_layout_passes=False,
                                             use_tc_tiling_on_sc=False),
        scratch_types=[
            pltpu.VMEM((BPW,), jnp.int32),
            pltpu.VMEM((BPW,), jnp.int32),
            pltpu.VMEM((CH, D), jnp.float32),
            pltpu.VMEM((CH, D), jnp.float32),
            pltpu.VMEM((CH, D), jnp.float32),
            pltpu.VMEM((CH, D), jnp.float32),
            pltpu.VMEM((BPW,), jnp.float32),
            pltpu.SemaphoreType.DMA,
            pltpu.SemaphoreType.DMA,
        ],
    )(_sc_body)
    return call(user.astype(jnp.int32), news.astype(jnp.int32),
                user_table, news_table)


def kernel(user, news, user_table, news_table):
    scores = _scores_and_news(user, news, user_table, news_table)
    one = lax.optimization_barrier(jnp.float32(1.0))
    nt = news_table * one
    ut = jnp.copy(user_table)
    return (ut, nt, scores)


# speedup vs baseline: 1.0902x; 1.0231x over previous
import functools

import jax
import jax.numpy as jnp
from jax import lax
from jax.experimental import pallas as pl
from jax.experimental.pallas import tpu as pltpu
from jax.experimental.pallas import tpu_sc as plsc

NC = 2
NS = 16
L = 16
NW = NC * NS

B = 16384
D = 128
BPW = B // NW
CH = 128
NCHUNK = BPW // CH

def _sc_body(user_ref, news_ref, ut_ref, nt_ref, scores_out,
             uidx, nidx, ubuf0, ubuf1, nbuf0, nbuf1, scores, usem, nsem):
    ubufs = (ubuf0, ubuf1)
    nbufs = (nbuf0, nbuf1)
    wid = lax.axis_index("s") * NC + lax.axis_index("c")
    base = wid * BPW

    for c in range(NCHUNK):
        pltpu.sync_copy(user_ref.at[pl.ds(base + c * CH, CH)], uidx.at[c])
        pltpu.sync_copy(news_ref.at[pl.ds(base + c * CH, CH)], nidx.at[c])

    uh = [None] * NCHUNK
    nh = [None] * NCHUNK
    uh[0] = pltpu.async_copy(ut_ref.at[uidx.at[0]], ubufs[0], usem)
    nh[0] = pltpu.async_copy(nt_ref.at[nidx.at[0]], nbufs[0], nsem)

    lanes = lax.iota(jnp.int32, L)

    for c in range(NCHUNK):
        cur = c % 2
        uh[c].wait()
        nh[c].wait()
        if c + 1 < NCHUNK:
            nxt = (c + 1) % 2
            uh[c + 1] = pltpu.async_copy(ut_ref.at[uidx.at[c + 1]], ubufs[nxt], usem)
            nh[c + 1] = pltpu.async_copy(nt_ref.at[nidx.at[c + 1]], nbufs[nxt], nsem)

        def grp_body(g, _, cur=cur, c=c):
            vec = jnp.zeros((L,), jnp.float32)
            for r in range(L):
                i = g * L + r
                acc = ubufs[cur][i, pl.ds(0, L)] * nbufs[cur][i, pl.ds(0, L)]
                for j in range(1, D // L):
                    acc = acc + (ubufs[cur][i, pl.ds(j * L, L)]
                                 * nbufs[cur][i, pl.ds(j * L, L)])
                s = jnp.sum(acc)
                vec = jnp.where(lanes == r, s, vec)
            scores[pl.ds(c * CH + g * L, L)] = vec
            return 0

        lax.fori_loop(0, CH // L, grp_body, 0)

    pltpu.sync_copy(scores, scores_out.at[pl.ds(base, BPW)])


@jax.jit
def _scores_on_sc(user, news, user_table, news_table):
    mesh = plsc.VectorSubcoreMesh(core_axis_name="c", subcore_axis_name="s",
                                  num_cores=NC, num_subcores=NS)
    call = functools.partial(
        pl.kernel,
        out_type=jax.ShapeDtypeStruct((B,), jnp.float32),
        mesh=mesh,
        compiler_params=pltpu.CompilerParams(needs_layout_passes=False,
                                             use_tc_tiling_on_sc=False),
        scratch_types=[
            pltpu.VMEM((NCHUNK, CH), jnp.int32),
            pltpu.VMEM((NCHUNK, CH), jnp.int32),
            pltpu.VMEM((CH, D), jnp.float32),
            pltpu.VMEM((CH, D), jnp.float32),
            pltpu.VMEM((CH, D), jnp.float32),
            pltpu.VMEM((CH, D), jnp.float32),
            pltpu.VMEM((BPW,), jnp.float32),
            pltpu.SemaphoreType.DMA,
            pltpu.SemaphoreType.DMA,
        ],
    )(_sc_body)
    return call(user.astype(jnp.int32), news.astype(jnp.int32),
                user_table, news_table)


def kernel(user, news, user_table, news_table):
    scores = _scores_on_sc(user, news, user_table, news_table)
    one = lax.optimization_barrier(jnp.float32(1.0))
    ut = user_table * one
    nt = news_table * one
    return (ut, nt, scores)
